# baseline (speedup 1.0000x reference)
import jax
import jax.numpy as jnp
from jax.experimental import pallas as pl
from jax.experimental.pallas import tpu as pltpu

N = 10000
E = 2000
D = 256
ALPHA = 0.2
BN = 1000
NB = N // BN
LOG2E = 1.4426950408889634


def _acc_kernel(x_ref, h_ref, w1_ref, w1a_ref, a1hi_ref, a1blo_ref,
                c1_ref, a1lo_ref,
                h8_ref, p1_ref, acc_ref, z_ref, sacc_s, sz_s):
    i = pl.program_id(0)
    f32 = jnp.float32
    bf16 = jnp.bfloat16
    dn = (((0,), (0,)), ((), ()))

    @pl.when(i == 0)
    def _():
        sacc_s[...] = jnp.zeros_like(sacc_s)
        sz_s[...] = jnp.zeros_like(sz_s)

    x = x_ref[...]
    h = h_ref[...]
    w1a = w1a_ref[...]
    v1a = jnp.dot(w1a, a1hi_ref[...], preferred_element_type=f32)
    v1b = jnp.dot(w1a, a1blo_ref[...], preferred_element_type=f32)
    c0 = jnp.sum(c1_ref[...] * a1lo_ref[...])
    s1v = jnp.dot(x, v1a, preferred_element_type=f32) + c0
    u = jnp.exp(jnp.maximum(s1v, ALPHA * s1v))
    p1 = jnp.dot(x, v1b, preferred_element_type=f32) * LOG2E
    p1_ref[0] = jax.lax.transpose(p1, (1, 0))
    h8_ref[...] = h.astype(jnp.int8)
    hb = h.astype(bf16)
    xt = jnp.dot(x, w1_ref[...], preferred_element_type=f32)
    t = u * xt
    th = t.astype(bf16)
    tl = (t - th.astype(f32)).astype(bf16)
    sacc_s[...] += (
        jax.lax.dot_general(th, hb, dn, preferred_element_type=f32)
        + jax.lax.dot_general(tl, hb, dn, preferred_element_type=f32))
    sz_s[...] += jax.lax.dot_general(u, h, dn, preferred_element_type=f32)

    @pl.when(i == NB - 1)
    def _():
        acc_ref[...] = sacc_s[...]
        z_ref[...] = sz_s[...]


def _node_kernel(h8_ref, p1_ref, acc_ref, z_ref, w1e_ref, w2_ref, w2e_ref,
                 w2a_ref, a1bhi_ref, a2bhi_ref, a2blo_ref,
                 node_ref, edge2_ref,
                 e1b_s, e2b_s, q1_s, q2_s, v2b_s):
    i = pl.program_id(0)
    f32 = jnp.float32
    bf16 = jnp.bfloat16
    dn = (((0,), (0,)), ((), ()))

    @pl.when(i == 0)
    def _():
        edge1t = acc_ref[...] * (1.0 / z_ref[...])
        e1b_s[...] = jax.lax.transpose(edge1t.astype(bf16), (1, 0))
        w1v = jnp.dot(w1e_ref[...], a1bhi_ref[...],
                      preferred_element_type=f32)
        q1_s[...] = jax.lax.dot_general(
            w1v, edge1t, dn, preferred_element_type=f32) * LOG2E
        edge2t = jax.lax.dot_general(w2_ref[...], edge1t, dn,
                                     preferred_element_type=f32)
        edge2_ref[...] = jax.lax.transpose(edge2t, (1, 0))
        e2b_s[...] = jax.lax.transpose(edge2t.astype(bf16), (1, 0))
        w2v = jnp.dot(w2e_ref[...], a2bhi_ref[...],
                      preferred_element_type=f32)
        q2_s[...] = jax.lax.dot_general(
            w2v, edge2t, dn, preferred_element_type=f32) * LOG2E
        v2b_s[...] = jnp.dot(w2a_ref[...], a2blo_ref[...],
                             preferred_element_type=f32) * LOG2E

    ones = jnp.ones((E, 1), bf16)
    hf = h8_ref[...].astype(f32)
    p1 = jax.lax.transpose(p1_ref[0], (1, 0))
    w = p1 + q1_s[...]
    e = (jnp.exp2(jnp.maximum(w, ALPHA * w)) * hf).astype(bf16)
    z1 = jnp.dot(e, ones, preferred_element_type=f32)
    node1 = jnp.dot(e, e1b_s[...], preferred_element_type=f32) * (1.0 / z1)
    p2 = jnp.dot(node1, v2b_s[...], preferred_element_type=f32)
    w2s = p2 + q2_s[...]
    e2 = (jnp.exp2(jnp.maximum(w2s, ALPHA * w2s)) * hf).astype(bf16)
    z2 = jnp.dot(e2, ones, preferred_element_type=f32)
    node_ref[...] = jnp.dot(e2, e2b_s[...],
                            preferred_element_type=f32) * (1.0 / z2)


def kernel(x, H, W1, W1a, W1e, a1, a1b, c1, W2, W2a, W2e, a2, a2b, c2):
    f32 = jnp.float32
    bf16 = jnp.bfloat16
    a1hi = a1[D:].reshape(D, 1)
    a1lo = a1[:D].reshape(1, D)
    a1blo = a1b[:D].reshape(D, 1)
    a1bhi = a1b[D:].reshape(D, 1)
    a2blo = a2b[:D].reshape(D, 1)
    a2bhi = a2b[D:].reshape(D, 1)
    c1r = c1.reshape(1, D)
    const = lambda i: (0, 0)

    h8, p1t, acc, z = pl.pallas_call(
        _acc_kernel,
        grid=(NB,),
        in_specs=[pl.BlockSpec((BN, D), lambda i: (i, 0)),
                  pl.BlockSpec((BN, E), lambda i: (i, 0)),
                  pl.BlockSpec((D, D), const),
                  pl.BlockSpec((D, D), const),
                  pl.BlockSpec((D, 1), const),
                  pl.BlockSpec((D, 1), const),
                  pl.BlockSpec((1, D), const),
                  pl.BlockSpec((1, D), const)],
        out_specs=(pl.BlockSpec((BN, E), lambda i: (i, 0)),
                   pl.BlockSpec((1, 1, BN), lambda i: (i, 0, 0)),
                   pl.BlockSpec((D, E), const),
                   pl.BlockSpec((1, E), const)),
        out_shape=(jax.ShapeDtypeStruct((N, E), jnp.int8),
                   jax.ShapeDtypeStruct((NB, 1, BN), f32),
                   jax.ShapeDtypeStruct((D, E), f32),
                   jax.ShapeDtypeStruct((1, E), f32)),
        scratch_shapes=[pltpu.VMEM((D, E), f32), pltpu.VMEM((1, E), f32)],
    )(x, H, W1, W1a, a1hi, a1blo, c1r, a1lo)

    node2, edge2 = pl.pallas_call(
        _node_kernel,
        grid=(NB,),
        in_specs=[pl.BlockSpec((BN, E), lambda i: (i, 0)),
                  pl.BlockSpec((1, 1, BN), lambda i: (i, 0, 0)),
                  pl.BlockSpec((D, E), const),
                  pl.BlockSpec((1, E), const),
                  pl.BlockSpec((D, D), const),
                  pl.BlockSpec((D, D), const),
                  pl.BlockSpec((D, D), const),
                  pl.BlockSpec((D, D), const),
                  pl.BlockSpec((D, 1), const),
                  pl.BlockSpec((D, 1), const),
                  pl.BlockSpec((D, 1), const)],
        out_specs=(pl.BlockSpec((BN, D), lambda i: (i, 0)),
                   pl.BlockSpec((E, D), const)),
        out_shape=(jax.ShapeDtypeStruct((N, D), f32),
                   jax.ShapeDtypeStruct((E, D), f32)),
        scratch_shapes=[pltpu.VMEM((E, D), bf16),
                        pltpu.VMEM((E, D), bf16),
                        pltpu.VMEM((1, E), f32),
                        pltpu.VMEM((1, E), f32),
                        pltpu.VMEM((D, 1), f32)],
    )(h8, p1t, acc, z, W1e, W2, W2e, W2a, a1bhi, a2bhi, a2blo)

    return (node2, edge2)
